# Initial kernel scaffold; baseline (speedup 1.0000x reference)
import functools
import jax
import jax.numpy as jnp
from jax import lax
from jax.experimental import pallas as pl
from jax.experimental.pallas import tpu as pltpu
from jax.experimental.pallas import tpu_sc as plsc

D = 128
HID = 256
N = 10000
E = 320000
EPS = 1e-5
NC = 2
NS = 16
NW = NC * NS
EPW = E // NW
CH = 128
NFULL = EPW // CH
TAIL = EPW - NFULL * CH
RPT = N // NS
RW = 125
SQRT_HALF = 0.7071067811865476


def _sc_mesh():
    return plsc.VectorSubcoreMesh(
        core_axis_name="c", subcore_axis_name="s", num_cores=NC, num_subcores=NS
    )


def _p1_body(edges_hbm, dst_hbm, msg_hbm, ssq_hbm,
             ebuf, ibuf, etail, itail, zbuf, obuf, msg_acc):
    c = lax.axis_index("c")
    s = lax.axis_index("s")
    wid = s * NC + c
    zeros = jnp.zeros((16,), jnp.float32)

    def zrow(r, carry):
        for g in range(8):
            zbuf[r, pl.ds(16 * g, 16)] = zeros
        return carry
    lax.fori_loop(0, RW, zrow, 0)
    for k in range(RPT // RW):
        pltpu.sync_copy(zbuf, msg_acc.at[pl.ds(s * RPT + k * RW, RW)])
    plsc.subcore_barrier()

    base0 = wid * EPW

    def chunk(i, acc):
        b = base0 + i * CH
        pltpu.sync_copy(dst_hbm.at[pl.ds(b, CH)], ibuf)
        pltpu.sync_copy(edges_hbm.at[pl.ds(b, CH)], ebuf)
        pltpu.sync_copy(ebuf, msg_acc.at[ibuf], add=True)

        def row(r, a):
            vs = [ebuf[r, pl.ds(16 * g, 16)] for g in range(8)]
            return tuple(a[g] + vs[g] * vs[g] for g in range(8))
        return lax.fori_loop(0, CH, row, acc)

    acc = lax.fori_loop(0, NFULL, chunk, tuple(zeros for _ in range(8)))

    bt = base0 + NFULL * CH
    pltpu.sync_copy(dst_hbm.at[pl.ds(bt, TAIL)], itail)
    pltpu.sync_copy(edges_hbm.at[pl.ds(bt, TAIL)], etail)
    pltpu.sync_copy(etail, msg_acc.at[itail], add=True)

    def trow(r, a):
        vs = [etail[r, pl.ds(16 * g, 16)] for g in range(8)]
        return tuple(a[g] + vs[g] * vs[g] for g in range(8))
    acc = lax.fori_loop(0, TAIL, trow, acc)

    for g in range(8):
        obuf[0, pl.ds(16 * g, 16)] = acc[g]
    pltpu.sync_copy(obuf.at[0], ssq_hbm.at[c, s])

    plsc.subcore_barrier()
    for k in range(RPT // RW):
        r0 = s * RPT + k * RW
        pltpu.sync_copy(msg_acc.at[pl.ds(r0, RW)], zbuf)
        pltpu.sync_copy(zbuf, msg_hbm.at[c, pl.ds(r0, RW)])


def _pass1(edges, dst):
    return pl.kernel(
        _p1_body,
        out_type=[
            jax.ShapeDtypeStruct((NC, N, D), jnp.float32),
            jax.ShapeDtypeStruct((NC, NS, D), jnp.float32),
        ],
        mesh=_sc_mesh(),
        scratch_types=[
            pltpu.VMEM((CH, D), jnp.float32),
            pltpu.VMEM((CH,), jnp.int32),
            pltpu.VMEM((TAIL, D), jnp.float32),
            pltpu.VMEM((TAIL,), jnp.int32),
            pltpu.VMEM((RW, D), jnp.float32),
            pltpu.VMEM((1, D), jnp.float32),
            pltpu.VMEM_SHARED((N, D), jnp.float32),
        ],
    )(edges, dst)


def _p2_body(nn_hbm, src_hbm, dst_hbm, g1_hbm, g2_hbm, st_hbm,
             g1v, g2v, g1t, g2t, isrc, idst, ist, idt, zbuf, sbuf, nn_sp):
    c = lax.axis_index("c")
    s = lax.axis_index("s")
    wid = s * NC + c
    zeros = jnp.zeros((16,), jnp.float32)

    for k in range(RPT // RW):
        r0 = s * RPT + k * RW
        pltpu.sync_copy(nn_hbm.at[pl.ds(r0, RW)], zbuf)
        pltpu.sync_copy(zbuf, nn_sp.at[pl.ds(r0, RW)])
    plsc.subcore_barrier()

    for t in range(4):
        for g in range(8):
            sbuf[t, pl.ds(16 * g, 16)] = zeros

    base0 = wid * EPW

    def accumulate(gv, rows, t):
        def row(r, a):
            vs = [gv[r, pl.ds(16 * g, 16)] for g in range(8)]
            return (tuple(a[g] + vs[g] for g in range(8))
                    + tuple(a[8 + g] + vs[g] * vs[g] for g in range(8)))
        a = lax.fori_loop(0, rows, row, tuple(zeros for _ in range(16)))
        for g in range(8):
            plsc.addupdate(sbuf.at[2 * t, pl.ds(16 * g, 16)], a[g])
            plsc.addupdate(sbuf.at[2 * t + 1, pl.ds(16 * g, 16)], a[8 + g])

    def chunk(i, carry):
        b = base0 + i * CH
        pltpu.sync_copy(src_hbm.at[pl.ds(b, CH)], isrc)
        pltpu.sync_copy(dst_hbm.at[pl.ds(b, CH)], idst)
        pltpu.sync_copy(nn_sp.at[isrc], g1v)
        pltpu.sync_copy(nn_sp.at[idst], g2v)
        pltpu.sync_copy(g1v, g1_hbm.at[pl.ds(b, CH)])
        pltpu.sync_copy(g2v, g2_hbm.at[pl.ds(b, CH)])
        accumulate(g1v, CH, 0)
        accumulate(g2v, CH, 1)
        return carry
    lax.fori_loop(0, NFULL, chunk, 0)

    bt = base0 + NFULL * CH
    pltpu.sync_copy(src_hbm.at[pl.ds(bt, TAIL)], ist)
    pltpu.sync_copy(dst_hbm.at[pl.ds(bt, TAIL)], idt)
    pltpu.sync_copy(nn_sp.at[ist], g1t)
    pltpu.sync_copy(nn_sp.at[idt], g2t)
    pltpu.sync_copy(g1t, g1_hbm.at[pl.ds(bt, TAIL)])
    pltpu.sync_copy(g2t, g2_hbm.at[pl.ds(bt, TAIL)])
    accumulate(g1t, TAIL, 0)
    accumulate(g2t, TAIL, 1)

    pltpu.sync_copy(sbuf, st_hbm.at[c, s])


def _pass2(nn, src, dst):
    return pl.kernel(
        _p2_body,
        out_type=[
            jax.ShapeDtypeStruct((E, D), jnp.float32),
            jax.ShapeDtypeStruct((E, D), jnp.float32),
            jax.ShapeDtypeStruct((NC, NS, 4, D), jnp.float32),
        ],
        mesh=_sc_mesh(),
        scratch_types=[
            pltpu.VMEM((CH, D), jnp.float32),
            pltpu.VMEM((CH, D), jnp.float32),
            pltpu.VMEM((TAIL, D), jnp.float32),
            pltpu.VMEM((TAIL, D), jnp.float32),
            pltpu.VMEM((CH,), jnp.int32),
            pltpu.VMEM((CH,), jnp.int32),
            pltpu.VMEM((TAIL,), jnp.int32),
            pltpu.VMEM((TAIL,), jnp.int32),
            pltpu.VMEM((RW, D), jnp.float32),
            pltpu.VMEM((4, D), jnp.float32),
            pltpu.VMEM_SHARED((N, D), jnp.float32),
        ],
    )(nn, src, dst)


def _gelu(x):
    return 0.5 * x * (1.0 + lax.erf(x * SQRT_HALF))


def _node_body(nodes_ref, m0_ref, m1_ref, w1a_ref, w1b_ref, b1_ref,
               w2_ref, b2_ref, ga_ref, gb_ref, ba_ref, bb_ref,
               nn_ref, cs_ref):
    nodes = nodes_ref[...]
    msg = m0_ref[...] + m1_ref[...]
    cs_ref[...] = jnp.sum(msg, axis=0, keepdims=True)

    mu1 = jnp.mean(nodes, axis=0, keepdims=True)
    v1 = jnp.mean(nodes * nodes, axis=0, keepdims=True) - mu1 * mu1
    mu2 = jnp.mean(msg, axis=0, keepdims=True)
    v2 = jnp.mean(msg * msg, axis=0, keepdims=True) - mu2 * mu2

    x1 = (nodes - mu1) / jnp.sqrt(v1 + EPS) * ga_ref[...] + ba_ref[...]
    x2 = (msg - mu2) / jnp.sqrt(v2 + EPS) * gb_ref[...] + bb_ref[...]

    h = (jnp.dot(x1, w1a_ref[...], preferred_element_type=jnp.float32)
         + jnp.dot(x2, w1b_ref[...], preferred_element_type=jnp.float32)
         + b1_ref[...])
    h = _gelu(h)
    nn_ref[...] = (jnp.dot(h, w2_ref[...], preferred_element_type=jnp.float32)
                   + b2_ref[...] + nodes)


def _node_update(nodes, msg0, msg1, W1, b1, W2, b2, gamma, beta):
    return pl.pallas_call(
        _node_body,
        out_shape=[
            jax.ShapeDtypeStruct((N, D), jnp.float32),
            jax.ShapeDtypeStruct((1, D), jnp.float32),
        ],
    )(nodes, msg0, msg1,
      W1[:D], W1[D:], b1.reshape(1, HID), W2, b2.reshape(1, D),
      gamma[:D].reshape(1, D), gamma[D:].reshape(1, D),
      beta[:D].reshape(1, D), beta[D:].reshape(1, D))


BLK = 1280
NBLK = E // BLK


def _edge_body(g1_ref, g2_ref, e_ref, st_ref, ssq_ref, cs_ref,
               ga_ref, be_ref, w1a_ref, w1b_ref, w1c_ref, b1_ref,
               w2_ref, b2_ref, out_ref):
    st = st_ref[...].reshape(NC * NS, 4 * D)
    stsum = jnp.sum(st, axis=0, keepdims=True)
    s1 = stsum[:, 0 * D:1 * D]
    q1 = stsum[:, 1 * D:2 * D]
    s2 = stsum[:, 2 * D:3 * D]
    q2 = stsum[:, 3 * D:4 * D]
    s3 = cs_ref[...]
    q3 = jnp.sum(ssq_ref[...].reshape(NC * NS, D), axis=0, keepdims=True)

    inv_e = 1.0 / E
    m1 = s1 * inv_e
    m2 = s2 * inv_e
    m3 = s3 * inv_e
    r1 = 1.0 / jnp.sqrt(q1 * inv_e - m1 * m1 + EPS)
    r2 = 1.0 / jnp.sqrt(q2 * inv_e - m2 * m2 + EPS)
    r3 = 1.0 / jnp.sqrt(q3 * inv_e - m3 * m3 + EPS)

    ga = ga_ref[...]
    be = be_ref[...]
    eb = e_ref[...]
    x1 = (g1_ref[...] - m1) * r1 * ga[:, :D] + be[:, :D]
    x2 = (g2_ref[...] - m2) * r2 * ga[:, D:2 * D] + be[:, D:2 * D]
    x3 = (eb - m3) * r3 * ga[:, 2 * D:] + be[:, 2 * D:]

    h = (jnp.dot(x1, w1a_ref[...], preferred_element_type=jnp.float32)
         + jnp.dot(x2, w1b_ref[...], preferred_element_type=jnp.float32)
         + jnp.dot(x3, w1c_ref[...], preferred_element_type=jnp.float32)
         + b1_ref[...])
    h = _gelu(h)
    out_ref[...] = (jnp.dot(h, w2_ref[...], preferred_element_type=jnp.float32)
                    + b2_ref[...] + eb)


def _edge_update(g1, g2, edges, st, ssq, cs, gamma, beta, W1, b1, W2, b2):
    blk = lambda: pl.BlockSpec((BLK, D), lambda i: (i, 0))
    full = lambda shape: pl.BlockSpec(shape, lambda i: tuple(0 for _ in shape))
    return pl.pallas_call(
        _edge_body,
        grid=(NBLK,),
        in_specs=[
            blk(), blk(), blk(),
            full((NC, NS, 4, D)), full((NC, NS, D)), full((1, D)),
            full((1, 3 * D)), full((1, 3 * D)),
            full((D, HID)), full((D, HID)), full((D, HID)), full((1, HID)),
            full((HID, D)), full((1, D)),
        ],
        out_specs=blk(),
        out_shape=jax.ShapeDtypeStruct((E, D), jnp.float32),
        compiler_params=pltpu.CompilerParams(
            dimension_semantics=("arbitrary",),
        ),
    )(g1, g2, edges, st, ssq, cs,
      gamma.reshape(1, 3 * D), beta.reshape(1, 3 * D),
      W1[:D], W1[D:2 * D], W1[2 * D:], b1.reshape(1, HID),
      W2, b2.reshape(1, D))


def kernel(nodes, edges, graph,
           node_W1, node_b1, node_W2, node_b2, node_gamma, node_beta,
           edge_W1, edge_b1, edge_W2, edge_b2, edge_gamma, edge_beta):
    src = graph[0]
    dst = graph[1]
    msg, ssq = _pass1(edges, dst)
    nn, cs = _node_update(nodes, msg[0], msg[1],
                          node_W1, node_b1, node_W2, node_b2,
                          node_gamma, node_beta)
    g1, g2, st = _pass2(nn, src, dst)
    new_edges = _edge_update(g1, g2, edges, st, ssq, cs,
                             edge_gamma, edge_beta,
                             edge_W1, edge_b1, edge_W2, edge_b2)
    return (nn, new_edges)


# speedup vs baseline: 3.6198x; 3.6198x over previous
import functools
import jax
import jax.numpy as jnp
from jax import lax
from jax.experimental import pallas as pl
from jax.experimental.pallas import tpu as pltpu
from jax.experimental.pallas import tpu_sc as plsc

D = 128
HID = 256
N = 10000
E = 320000
EPS = 1e-5
NC = 2
NS = 16
NW = NC * NS
EPW = E // NW
CH = 128
NFULL = EPW // CH
TAIL = EPW - NFULL * CH
N_PAD = 10240
RPT = N_PAD // NS
RW = 128
SQRT_HALF = 0.7071067811865476


def _sc_mesh():
    return plsc.VectorSubcoreMesh(
        core_axis_name="c", subcore_axis_name="s", num_cores=NC, num_subcores=NS
    )


def _p1_body(edges_hbm, dst_hbm, msg_hbm, ssq_hbm,
             ebuf, ibuf, etail, itail, zbuf, obuf, msg_acc):
    c = lax.axis_index("c")
    s = lax.axis_index("s")
    wid = s * NC + c
    zeros = jnp.zeros((16,), jnp.float32)

    def zrow(r, carry):
        for g in range(8):
            zbuf[r, pl.ds(16 * g, 16)] = zeros
        return carry
    lax.fori_loop(0, RW, zrow, 0)
    for k in range(RPT // RW):
        pltpu.sync_copy(zbuf, msg_acc.at[pl.ds(s * RPT + k * RW, RW)])
    plsc.subcore_barrier()

    base0 = wid * EPW

    def chunk(i, acc):
        b = base0 + i * CH
        pltpu.sync_copy(dst_hbm.at[pl.ds(b, CH)], ibuf)
        pltpu.sync_copy(edges_hbm.at[pl.ds(b, CH)], ebuf)
        pltpu.sync_copy(ebuf, msg_acc.at[ibuf], add=True)

        def row(r, a):
            vs = [ebuf[r, pl.ds(16 * g, 16)] for g in range(8)]
            return tuple(a[g] + vs[g] * vs[g] for g in range(8))
        return lax.fori_loop(0, CH, row, acc)

    acc = lax.fori_loop(0, NFULL, chunk, tuple(zeros for _ in range(8)))

    bt = base0 + NFULL * CH
    pltpu.sync_copy(dst_hbm.at[pl.ds(bt, TAIL)], itail)
    pltpu.sync_copy(edges_hbm.at[pl.ds(bt, TAIL)], etail)
    pltpu.sync_copy(etail, msg_acc.at[itail], add=True)

    def trow(r, a):
        vs = [etail[r, pl.ds(16 * g, 16)] for g in range(8)]
        return tuple(a[g] + vs[g] * vs[g] for g in range(8))
    acc = lax.fori_loop(0, TAIL, trow, acc)

    for g in range(8):
        obuf[pl.ds(16 * g, 16)] = acc[g]
    pltpu.sync_copy(obuf, ssq_hbm.at[pl.ds((c * NS + s) * D, D)])

    plsc.subcore_barrier()
    for k in range(RPT // RW):
        r0 = s * RPT + k * RW
        pltpu.sync_copy(msg_acc.at[pl.ds(r0, RW)], zbuf)
        pltpu.sync_copy(zbuf, msg_hbm.at[c, pl.ds(r0, RW)])


def _pass1(edges, dst):
    return pl.kernel(
        _p1_body,
        out_type=[
            jax.ShapeDtypeStruct((NC, N_PAD, D), jnp.float32),
            jax.ShapeDtypeStruct((NC * NS * D,), jnp.float32),
        ],
        mesh=_sc_mesh(),
        scratch_types=[
            pltpu.VMEM((CH, D), jnp.float32),
            pltpu.VMEM((CH,), jnp.int32),
            pltpu.VMEM((TAIL, D), jnp.float32),
            pltpu.VMEM((TAIL,), jnp.int32),
            pltpu.VMEM((RW, D), jnp.float32),
            pltpu.VMEM((D,), jnp.float32),
            pltpu.VMEM_SHARED((N_PAD, D), jnp.float32),
        ],
    )(edges, dst)


def _p2_body(nn_hbm, src_hbm, dst_hbm, g1_hbm, g2_hbm, st_hbm,
             g1v, g2v, g1t, g2t, isrc, idst, ist, idt, sbuf, nn_sp):
    c = lax.axis_index("c")
    s = lax.axis_index("s")
    wid = s * NC + c
    zeros = jnp.zeros((16,), jnp.float32)

    for k in range(RPT // RW):
        r0 = s * RPT + k * RW
        pltpu.sync_copy(nn_hbm.at[pl.ds(r0, RW)], g1v)
        pltpu.sync_copy(g1v, nn_sp.at[pl.ds(r0, RW)])
    plsc.subcore_barrier()

    for t in range(4):
        for g in range(8):
            sbuf[pl.ds(t * D + 16 * g, 16)] = zeros

    base0 = wid * EPW

    def accumulate(gv, rows, t):
        def row(r, a):
            vs = [gv[r, pl.ds(16 * g, 16)] for g in range(8)]
            return (tuple(a[g] + vs[g] for g in range(8))
                    + tuple(a[8 + g] + vs[g] * vs[g] for g in range(8)))
        a = lax.fori_loop(0, rows, row, tuple(zeros for _ in range(16)))
        for g in range(8):
            plsc.addupdate(sbuf.at[pl.ds(2 * t * D + 16 * g, 16)], a[g])
            plsc.addupdate(sbuf.at[pl.ds((2 * t + 1) * D + 16 * g, 16)], a[8 + g])

    def chunk(i, carry):
        b = base0 + i * CH
        pltpu.sync_copy(src_hbm.at[pl.ds(b, CH)], isrc)
        pltpu.sync_copy(dst_hbm.at[pl.ds(b, CH)], idst)
        pltpu.sync_copy(nn_sp.at[isrc], g1v)
        pltpu.sync_copy(nn_sp.at[idst], g2v)
        pltpu.sync_copy(g1v, g1_hbm.at[pl.ds(b, CH)])
        pltpu.sync_copy(g2v, g2_hbm.at[pl.ds(b, CH)])
        accumulate(g1v, CH, 0)
        accumulate(g2v, CH, 1)
        return carry
    lax.fori_loop(0, NFULL, chunk, 0)

    bt = base0 + NFULL * CH
    pltpu.sync_copy(src_hbm.at[pl.ds(bt, TAIL)], ist)
    pltpu.sync_copy(dst_hbm.at[pl.ds(bt, TAIL)], idt)
    pltpu.sync_copy(nn_sp.at[ist], g1t)
    pltpu.sync_copy(nn_sp.at[idt], g2t)
    pltpu.sync_copy(g1t, g1_hbm.at[pl.ds(bt, TAIL)])
    pltpu.sync_copy(g2t, g2_hbm.at[pl.ds(bt, TAIL)])
    accumulate(g1t, TAIL, 0)
    accumulate(g2t, TAIL, 1)

    pltpu.sync_copy(sbuf, st_hbm.at[pl.ds((c * NS + s) * 4 * D, 4 * D)])


def _pass2(nn, src, dst):
    return pl.kernel(
        _p2_body,
        out_type=[
            jax.ShapeDtypeStruct((E, D), jnp.float32),
            jax.ShapeDtypeStruct((E, D), jnp.float32),
            jax.ShapeDtypeStruct((NC * NS * 4 * D,), jnp.float32),
        ],
        mesh=_sc_mesh(),
        scratch_types=[
            pltpu.VMEM((CH, D), jnp.float32),
            pltpu.VMEM((CH, D), jnp.float32),
            pltpu.VMEM((TAIL, D), jnp.float32),
            pltpu.VMEM((TAIL, D), jnp.float32),
            pltpu.VMEM((CH,), jnp.int32),
            pltpu.VMEM((CH,), jnp.int32),
            pltpu.VMEM((TAIL,), jnp.int32),
            pltpu.VMEM((TAIL,), jnp.int32),
            pltpu.VMEM((4 * D,), jnp.float32),
            pltpu.VMEM_SHARED((N_PAD, D), jnp.float32),
        ],
    )(nn, src, dst)


def _gelu(x):
    return 0.5 * x * (1.0 + lax.erf(x * SQRT_HALF))


def _node_body(nodes_ref, m0_ref, m1_ref, w1a_ref, w1b_ref, b1_ref,
               w2_ref, b2_ref, ga_ref, gb_ref, ba_ref, bb_ref,
               nn_ref, cs_ref):
    nodes = nodes_ref[...]
    msg = m0_ref[...] + m1_ref[...]
    cs_ref[...] = jnp.sum(msg, axis=0, keepdims=True)

    mu1 = jnp.mean(nodes, axis=0, keepdims=True)
    v1 = jnp.mean(nodes * nodes, axis=0, keepdims=True) - mu1 * mu1
    mu2 = jnp.mean(msg, axis=0, keepdims=True)
    v2 = jnp.mean(msg * msg, axis=0, keepdims=True) - mu2 * mu2

    x1 = (nodes - mu1) / jnp.sqrt(v1 + EPS) * ga_ref[...] + ba_ref[...]
    x2 = (msg - mu2) / jnp.sqrt(v2 + EPS) * gb_ref[...] + bb_ref[...]

    h = (jnp.dot(x1, w1a_ref[...], preferred_element_type=jnp.float32)
         + jnp.dot(x2, w1b_ref[...], preferred_element_type=jnp.float32)
         + b1_ref[...])
    h = _gelu(h)
    nn_ref[...] = (jnp.dot(h, w2_ref[...], preferred_element_type=jnp.float32)
                   + b2_ref[...] + nodes)


def _node_update(nodes, msg0, msg1, W1, b1, W2, b2, gamma, beta):
    return pl.pallas_call(
        _node_body,
        out_shape=[
            jax.ShapeDtypeStruct((N, D), jnp.float32),
            jax.ShapeDtypeStruct((1, D), jnp.float32),
        ],
    )(nodes, msg0, msg1,
      W1[:D], W1[D:], b1.reshape(1, HID), W2, b2.reshape(1, D),
      gamma[:D].reshape(1, D), gamma[D:].reshape(1, D),
      beta[:D].reshape(1, D), beta[D:].reshape(1, D))


BLK = 1280
NBLK = E // BLK


def _edge_body(g1_ref, g2_ref, e_ref, st_ref, ssq_ref, cs_ref,
               ga_ref, be_ref, w1a_ref, w1b_ref, w1c_ref, b1_ref,
               w2_ref, b2_ref, out_ref):
    st = st_ref[...].reshape(NC * NS, 4 * D)
    stsum = jnp.sum(st, axis=0, keepdims=True)
    s1 = stsum[:, 0 * D:1 * D]
    q1 = stsum[:, 1 * D:2 * D]
    s2 = stsum[:, 2 * D:3 * D]
    q2 = stsum[:, 3 * D:4 * D]
    s3 = cs_ref[...]
    q3 = jnp.sum(ssq_ref[...].reshape(NC * NS, D), axis=0, keepdims=True)

    inv_e = 1.0 / E
    m1 = s1 * inv_e
    m2 = s2 * inv_e
    m3 = s3 * inv_e
    r1 = 1.0 / jnp.sqrt(q1 * inv_e - m1 * m1 + EPS)
    r2 = 1.0 / jnp.sqrt(q2 * inv_e - m2 * m2 + EPS)
    r3 = 1.0 / jnp.sqrt(q3 * inv_e - m3 * m3 + EPS)

    ga = ga_ref[...]
    be = be_ref[...]
    eb = e_ref[...]
    x1 = (g1_ref[...] - m1) * r1 * ga[:, :D] + be[:, :D]
    x2 = (g2_ref[...] - m2) * r2 * ga[:, D:2 * D] + be[:, D:2 * D]
    x3 = (eb - m3) * r3 * ga[:, 2 * D:] + be[:, 2 * D:]

    h = (jnp.dot(x1, w1a_ref[...], preferred_element_type=jnp.float32)
         + jnp.dot(x2, w1b_ref[...], preferred_element_type=jnp.float32)
         + jnp.dot(x3, w1c_ref[...], preferred_element_type=jnp.float32)
         + b1_ref[...])
    h = _gelu(h)
    out_ref[...] = (jnp.dot(h, w2_ref[...], preferred_element_type=jnp.float32)
                    + b2_ref[...] + eb)


def _edge_update(g1, g2, edges, st, ssq, cs, gamma, beta, W1, b1, W2, b2):
    blk = lambda: pl.BlockSpec((BLK, D), lambda i: (i, 0))
    full = lambda shape: pl.BlockSpec(shape, lambda i: tuple(0 for _ in shape))
    return pl.pallas_call(
        _edge_body,
        grid=(NBLK,),
        in_specs=[
            blk(), blk(), blk(),
            full((NC, NS, 4, D)), full((NC, NS, D)), full((1, D)),
            full((1, 3 * D)), full((1, 3 * D)),
            full((D, HID)), full((D, HID)), full((D, HID)), full((1, HID)),
            full((HID, D)), full((1, D)),
        ],
        out_specs=blk(),
        out_shape=jax.ShapeDtypeStruct((E, D), jnp.float32),
        compiler_params=pltpu.CompilerParams(
            dimension_semantics=("arbitrary",),
        ),
    )(g1, g2, edges, st, ssq, cs,
      gamma.reshape(1, 3 * D), beta.reshape(1, 3 * D),
      W1[:D], W1[D:2 * D], W1[2 * D:], b1.reshape(1, HID),
      W2, b2.reshape(1, D))


def kernel(nodes, edges, graph,
           node_W1, node_b1, node_W2, node_b2, node_gamma, node_beta,
           edge_W1, edge_b1, edge_W2, edge_b2, edge_gamma, edge_beta):
    src = graph[0]
    dst = graph[1]
    msg, ssq = _pass1(edges, dst)
    nn, cs = _node_update(nodes, msg[0, :N], msg[1, :N],
                          node_W1, node_b1, node_W2, node_b2,
                          node_gamma, node_beta)
    nn_pad = jnp.concatenate([nn, jnp.zeros((N_PAD - N, D), jnp.float32)])
    g1, g2, st = _pass2(nn_pad, src, dst)
    new_edges = _edge_update(g1, g2, edges,
                             st.reshape(NC, NS, 4, D), ssq.reshape(NC, NS, D),
                             cs, edge_gamma, edge_beta,
                             edge_W1, edge_b1, edge_W2, edge_b2)
    return (nn, new_edges)
